# baseline (speedup 1.0000x reference)
import jax
import jax.numpy as jnp
from jax import lax
from jax.experimental import pallas as pl
from jax.experimental.pallas import tpu as pltpu
from jax.experimental.pallas import tpu_sc as plsc

T = 1024
D = 768
NS = D // 128
B = 4
BLK = 128
CH = 32
LIST = T + CH + 32


def _route_body(h_ref, *out_refs):
    ssrc_refs = out_refs[:NS]
    sdst_refs = out_refs[NS : 2 * NS]
    idx_ref = out_refs[2 * NS]
    h3 = h_ref[0].reshape(T, 2, D)
    src = h3[:, 0, :]
    dst = h3[:, 1, :]
    sn = src / jnp.sqrt(jnp.sum(src * src, axis=1, keepdims=True))
    dn = dst / jnp.sqrt(jnp.sum(dst * dst, axis=1, keepdims=True))
    scores = lax.dot_general(
        sn, dn, (((1,), (1,)), ((), ())), preferred_element_type=jnp.float32
    )
    m_col = jnp.max(scores, axis=1, keepdims=True)
    jj = lax.broadcasted_iota(jnp.int32, (T, T), 1)
    idx_col = jnp.min(
        jnp.where(scores == m_col, jj, jnp.int32(2**30)), axis=1, keepdims=True
    )
    ii_col = lax.broadcasted_iota(jnp.int32, (T, 1), 0)
    idx_col = jnp.where(ii_col == 0, 0, idx_col)
    e = ((jj == idx_col) & (ii_col != 0)).astype(jnp.float32)
    counts_row = jnp.sum(e, axis=0, keepdims=True)
    recip_row = 1.0 / (1.0 + counts_row)
    recip_i_col = lax.dot_general(
        e, recip_row, (((1,), (1,)), ((), ())), preferred_element_type=jnp.float32
    )
    jj0 = lax.broadcasted_iota(jnp.int32, (T, T), 0)
    ident = (jj0 == jj).astype(jnp.float32)
    recip_col = lax.dot_general(
        ident, recip_row, (((1,), (1,)), ((), ())), preferred_element_type=jnp.float32
    )
    ssrc = src * recip_i_col
    sdst = dst * recip_col
    for c in range(NS):
        ssrc_refs[c][0] = ssrc[:, c * 128 : (c + 1) * 128]
        sdst_refs[c][0] = sdst[:, c * 128 : (c + 1) * 128]
    idx_ref[0] = idx_col


def _route_tc(h):
    slice_spec = pl.BlockSpec((1, T, 128), lambda b: (b, 0, 0))
    slice_shape = jax.ShapeDtypeStruct((B, T, 128), jnp.float32)
    return pl.pallas_call(
        _route_body,
        grid=(B,),
        in_specs=[pl.BlockSpec((1, 2 * T, D), lambda b: (b, 0, 0))],
        out_specs=[slice_spec] * (2 * NS)
        + [pl.BlockSpec((1, T, 1), lambda b: (b, 0, 0))],
        out_shape=[slice_shape] * (2 * NS)
        + [jax.ShapeDtypeStruct((B, T, 1), jnp.int32)],
    )(h)


def _sc_merge_body(*refs):
    ssrc = refs[:NS]
    sdst = refs[NS : 2 * NS]
    idx_hbm = refs[2 * NS]
    outs = refs[2 * NS + 1 : 3 * NS + 1]
    idx_v, srclist, dstlist, rowbuf, acc = refs[3 * NS + 1 :]

    c = lax.axis_index("c")
    s = lax.axis_index("s")
    b = 2 * c + s // 8
    jlo = (s % 8) * BLK

    for u in range(NS):
        pltpu.sync_copy(sdst[u].at[b, pl.ds(jlo, BLK)], acc.at[pl.ds(u * BLK, BLK)])

    pltpu.sync_copy(idx_hbm.at[b], idx_v.at[pl.ds(0, T)])

    zer16 = jnp.zeros((16,), jnp.int32)

    def compact(k, n):
        v16 = idx_v[pl.ds(k * 16, 16)]
        for l in range(16):
            vv = v16[l]
            srclist[pl.ds(n, 16)] = zer16 + (k * 16 + l)
            dstlist[pl.ds(n, 16)] = zer16 + (vv - jlo)
            match = (vv >= jlo) & (vv < jlo + BLK)
            n = n + match.astype(jnp.int32)
        return n

    n = lax.fori_loop(0, T // 16, compact, jnp.int32(0))

    def pad(k, _):
        srclist[pl.ds(n + k * 16, 16)] = zer16
        dstlist[pl.ds(n + k * 16, 16)] = zer16
        return 0

    lax.fori_loop(0, CH // 16 + 1, pad, 0)

    def chunk(q, _):
        for u in range(NS):
            pltpu.sync_copy(
                ssrc[u].at[b].at[srclist.at[pl.ds(q * CH, CH)]],
                rowbuf.at[pl.ds(u * CH, CH)],
            )

        def accrow(r, _2):
            dj = dstlist[pl.ds(q * CH + r, 16)][0]
            for u in range(NS):
                for g in range(8):
                    plsc.addupdate(
                        acc.at[u * BLK + dj, pl.ds(g * 16, 16)],
                        rowbuf[u * CH + r, pl.ds(g * 16, 16)],
                    )
            return 0

        lax.fori_loop(0, CH, accrow, 0)
        return 0

    nchunks = (n + (CH - 1)) // CH
    lax.fori_loop(0, nchunks, chunk, 0)

    for u in range(NS):
        pltpu.sync_copy(acc.at[pl.ds(u * BLK, BLK)], outs[u].at[b, pl.ds(jlo, BLK)])


def _merge_sc(ssrc, sdst, idx):
    mesh = plsc.VectorSubcoreMesh(core_axis_name="c", subcore_axis_name="s")
    f = pl.kernel(
        _sc_merge_body,
        out_type=[jax.ShapeDtypeStruct((B, T, 128), jnp.float32)] * NS,
        mesh=mesh,
        scratch_types=[
            pltpu.VMEM((T + 16,), jnp.int32),
            pltpu.VMEM((LIST,), jnp.int32),
            pltpu.VMEM((LIST,), jnp.int32),
            pltpu.VMEM((NS * CH, 128), jnp.float32),
            pltpu.VMEM((NS * BLK, 128), jnp.float32),
        ],
    )
    return f(*ssrc, *sdst, idx)


def kernel(hidden_states, attention_mask, self_attention_scores):
    Bh, L, Dd = hidden_states.shape
    assert Bh == B and L == 2 * T and Dd == D
    outs = _route_tc(hidden_states)
    ssrc, sdst, idx = outs[:NS], outs[NS : 2 * NS], outs[2 * NS]
    merged = _merge_sc(ssrc, sdst, idx.reshape(B, T))
    dst_m = jnp.concatenate(merged, axis=2)
    preserved = jnp.concatenate([hidden_states[:, :1, :], dst_m], axis=1)
    mask = jnp.zeros((B, 1, 1, T + 1), dtype=self_attention_scores.dtype)
    return preserved, mask


# speedup vs baseline: 1.0171x; 1.0171x over previous
import jax
import jax.numpy as jnp
from jax import lax
from jax.experimental import pallas as pl
from jax.experimental.pallas import tpu as pltpu
from jax.experimental.pallas import tpu_sc as plsc

T = 1024
D = 768
NS = D // 128
B = 4
BLK = 128
CH = 16
LIST = T + CH + 32


def _route_body(h_ref, ssrc_ref, sdst_ref, idx_ref):
    h3 = h_ref[0].reshape(T, 2, D)
    src = h3[:, 0, :]
    dst = h3[:, 1, :]
    sn = src / jnp.sqrt(jnp.sum(src * src, axis=1, keepdims=True))
    dn = dst / jnp.sqrt(jnp.sum(dst * dst, axis=1, keepdims=True))
    scores = lax.dot_general(
        sn, dn, (((1,), (1,)), ((), ())), preferred_element_type=jnp.float32
    )
    m_col = jnp.max(scores, axis=1, keepdims=True)
    jj = lax.broadcasted_iota(jnp.int32, (T, T), 1)
    idx_col = jnp.min(
        jnp.where(scores == m_col, jj, jnp.int32(2**30)), axis=1, keepdims=True
    )
    ii_col = lax.broadcasted_iota(jnp.int32, (T, 1), 0)
    idx_col = jnp.where(ii_col == 0, 0, idx_col)
    e = ((jj == idx_col) & (ii_col != 0)).astype(jnp.float32)
    counts_row = jnp.sum(e, axis=0, keepdims=True)
    recip_row = 1.0 / (1.0 + counts_row)
    recip_i_col = lax.dot_general(
        e, recip_row, (((1,), (1,)), ((), ())), preferred_element_type=jnp.float32
    )
    jj0 = lax.broadcasted_iota(jnp.int32, (T, T), 0)
    ident = (jj0 == jj).astype(jnp.float32)
    recip_col = lax.dot_general(
        ident, recip_row, (((1,), (1,)), ((), ())), preferred_element_type=jnp.float32
    )
    ssrc = src * recip_i_col
    sdst = dst * recip_col
    ssrc_ref[0] = jnp.concatenate(
        [ssrc[:, u * 128 : (u + 1) * 128] for u in range(NS)], axis=0
    )
    sdst_ref[0] = sdst
    idx_ref[0] = idx_col


def _route_tc(h):
    return pl.pallas_call(
        _route_body,
        grid=(B,),
        in_specs=[pl.BlockSpec((1, 2 * T, D), lambda b: (b, 0, 0))],
        out_specs=[
            pl.BlockSpec((1, NS * T, 128), lambda b: (b, 0, 0)),
            pl.BlockSpec((1, T, D), lambda b: (b, 0, 0)),
            pl.BlockSpec((1, T, 1), lambda b: (b, 0, 0)),
        ],
        out_shape=[
            jax.ShapeDtypeStruct((B, NS * T, 128), jnp.float32),
            jax.ShapeDtypeStruct((B, T, D), jnp.float32),
            jax.ShapeDtypeStruct((B, T, 1), jnp.int32),
        ],
    )(h)


def _sc_merge_body(
    ssrc_hbm, idx_hbm, out_hbm, idx_v, srclist, dstlist, piecelist, rowbuf, acc
):
    c = lax.axis_index("c")
    s = lax.axis_index("s")
    b = 2 * c + s // 8
    jlo = (s % 8) * BLK

    zf16 = jnp.zeros((16,), jnp.float32)

    def zero(k, _):
        for g in range(8):
            acc[k, pl.ds(g * 16, 16)] = zf16
        return 0

    lax.fori_loop(0, NS * BLK, zero, 0)

    pltpu.sync_copy(idx_hbm.at[b], idx_v.at[pl.ds(0, T)])

    zer16 = jnp.zeros((16,), jnp.int32)

    def compact(k, n):
        v16 = idx_v[pl.ds(k * 16, 16)]
        for l in range(16):
            vv = v16[l]
            srclist[pl.ds(n, 16)] = zer16 + (k * 16 + l)
            dstlist[pl.ds(n, 16)] = zer16 + (vv - jlo)
            match = (vv >= jlo) & (vv < jlo + BLK)
            n = n + match.astype(jnp.int32)
        return n

    n = lax.fori_loop(0, T // 16, compact, jnp.int32(0))

    def pad(k, _):
        srclist[pl.ds(n + k * 16, 16)] = zer16
        dstlist[pl.ds(n + k * 16, 16)] = zer16
        return 0

    lax.fori_loop(0, CH // 16 + 1, pad, 0)

    nchunks = (n + (CH - 1)) // CH

    def expand(k, _):
        q, w = k // (CH // 16), k % (CH // 16)
        s16 = srclist[pl.ds(q * CH + w * 16, 16)]
        for u in range(NS):
            piecelist[pl.ds(q * (NS * CH) + u * CH + w * 16, 16)] = s16 + u * T
        return 0

    lax.fori_loop(0, nchunks * (CH // 16), expand, 0)

    def chunk(q, _):
        pltpu.sync_copy(
            ssrc_hbm.at[b].at[piecelist.at[pl.ds(q * (NS * CH), NS * CH)]], rowbuf
        )

        def accrow(r, _2):
            dj = dstlist[pl.ds(q * CH + r, 16)][0]
            for u in range(NS):
                for g in range(8):
                    plsc.addupdate(
                        acc.at[u * BLK + dj, pl.ds(g * 16, 16)],
                        rowbuf[u * CH + r, pl.ds(g * 16, 16)],
                    )
            return 0

        lax.fori_loop(0, CH, accrow, 0)
        return 0

    lax.fori_loop(0, nchunks, chunk, 0)

    for u in range(NS):
        pltpu.sync_copy(
            acc.at[pl.ds(u * BLK, BLK)], out_hbm.at[b, pl.ds(u * T + jlo, BLK)]
        )


def _merge_sc(ssrc, idx):
    mesh = plsc.VectorSubcoreMesh(core_axis_name="c", subcore_axis_name="s")
    f = pl.kernel(
        _sc_merge_body,
        out_type=jax.ShapeDtypeStruct((B, NS * T, 128), jnp.float32),
        mesh=mesh,
        scratch_types=[
            pltpu.VMEM((T + 16,), jnp.int32),
            pltpu.VMEM((LIST,), jnp.int32),
            pltpu.VMEM((LIST,), jnp.int32),
            pltpu.VMEM((NS * (T + CH),), jnp.int32),
            pltpu.VMEM((NS * CH, 128), jnp.float32),
            pltpu.VMEM((NS * BLK, 128), jnp.float32),
        ],
    )
    return f(ssrc, idx)


def kernel(hidden_states, attention_mask, self_attention_scores):
    Bh, L, Dd = hidden_states.shape
    assert Bh == B and L == 2 * T and Dd == D
    ssrc, sdst, idx = _route_tc(hidden_states)
    sums = _merge_sc(ssrc, idx.reshape(B, T))
    sums = sums.reshape(B, NS, T, 128).transpose(0, 2, 1, 3).reshape(B, T, D)
    dst_m = sdst + sums
    preserved = jnp.concatenate([hidden_states[:, :1, :], dst_m], axis=1)
    mask = jnp.zeros((B, 1, 1, T + 1), dtype=self_attention_scores.dtype)
    return preserved, mask


# speedup vs baseline: 1.1925x; 1.1724x over previous
import jax
import jax.numpy as jnp
from jax import lax
from jax.experimental import pallas as pl
from jax.experimental.pallas import tpu as pltpu
from jax.experimental.pallas import tpu_sc as plsc

T = 1024
D = 768
B = 4
BLK = 128
CH = 32
LIST = T + CH + 32


def _route_body(h_ref, ssrc_ref, sdst_ref, idx_ref):
    h3 = h_ref[0].reshape(T, 2, D)
    src = h3[:, 0, :]
    dst = h3[:, 1, :]
    sn = src / jnp.sqrt(jnp.sum(src * src, axis=1, keepdims=True))
    dn = dst / jnp.sqrt(jnp.sum(dst * dst, axis=1, keepdims=True))
    scores = lax.dot_general(
        sn, dn, (((1,), (1,)), ((), ())), preferred_element_type=jnp.float32
    )
    m_col = jnp.max(scores, axis=1, keepdims=True)
    jj = lax.broadcasted_iota(jnp.int32, (T, T), 1)
    idx_col = jnp.min(
        jnp.where(scores == m_col, jj, jnp.int32(2**30)), axis=1, keepdims=True
    )
    ii_col = lax.broadcasted_iota(jnp.int32, (T, 1), 0)
    idx_col = jnp.where(ii_col == 0, 0, idx_col)
    e = ((jj == idx_col) & (ii_col != 0)).astype(jnp.float32)
    counts_row = jnp.sum(e, axis=0, keepdims=True)
    recip_row = 1.0 / (1.0 + counts_row)
    recip_i_col = lax.dot_general(
        e, recip_row, (((1,), (1,)), ((), ())), preferred_element_type=jnp.float32
    )
    jj0 = lax.broadcasted_iota(jnp.int32, (T, T), 0)
    ident = (jj0 == jj).astype(jnp.float32)
    recip_col = lax.dot_general(
        ident, recip_row, (((1,), (1,)), ((), ())), preferred_element_type=jnp.float32
    )
    ssrc_ref[0] = src * recip_i_col
    sdst_ref[0] = dst * recip_col
    idx_ref[0] = idx_col


def _route_tc(h):
    return pl.pallas_call(
        _route_body,
        grid=(B,),
        in_specs=[
            pl.BlockSpec((1, 2 * T, D), lambda b: (b, 0, 0)),
        ],
        out_specs=[
            pl.BlockSpec((1, T, D), lambda b: (b, 0, 0)),
            pl.BlockSpec((1, T, D), lambda b: (b, 0, 0)),
            pl.BlockSpec((1, T, 1), lambda b: (b, 0, 0)),
        ],
        out_shape=[
            jax.ShapeDtypeStruct((B, T, D), jnp.float32),
            jax.ShapeDtypeStruct((B, T, D), jnp.float32),
            jax.ShapeDtypeStruct((B, T, 1), jnp.int32),
        ],
    )(h)


def _sc_merge_body(
    ssrc_hbm, sdst_hbm, idx_hbm, out_hbm,
    idx_v, srclist, dstlist, rowbuf, acc,
):
    c = lax.axis_index("c")
    s = lax.axis_index("s")
    b = 2 * c + s // 8
    jlo = (s % 8) * BLK

    pltpu.sync_copy(sdst_hbm.at[b, pl.ds(jlo, BLK)], acc)

    pltpu.sync_copy(idx_hbm.at[b], idx_v.at[pl.ds(0, T)])

    zer16 = jnp.zeros((16,), jnp.int32)

    def compact(k, n):
        v16 = idx_v[pl.ds(k * 16, 16)]
        for l in range(16):
            vv = v16[l]
            srclist[pl.ds(n, 16)] = zer16 + (k * 16 + l)
            dstlist[pl.ds(n, 16)] = zer16 + (vv - jlo)
            match = (vv >= jlo) & (vv < jlo + BLK)
            n = n + match.astype(jnp.int32)
        return n

    n = lax.fori_loop(0, T // 16, compact, jnp.int32(0))

    zeros16 = jnp.zeros((16,), jnp.int32)

    def pad(k, _):
        srclist[pl.ds(n + k * 16, 16)] = zeros16
        dstlist[pl.ds(n + k * 16, 16)] = zeros16
        return 0

    lax.fori_loop(0, CH // 16 + 1, pad, 0)

    def chunk(q, _):
        pltpu.sync_copy(ssrc_hbm.at[b].at[srclist.at[pl.ds(q * CH, CH)]], rowbuf)

        def accrow(r, _2):
            dj = dstlist[pl.ds(q * CH + r, 16)][0]
            for g in range(D // 16):
                plsc.addupdate(
                    acc.at[dj, pl.ds(g * 16, 16)], rowbuf[r, pl.ds(g * 16, 16)]
                )
            return 0

        lax.fori_loop(0, CH, accrow, 0)
        return 0

    nchunks = (n + (CH - 1)) // CH
    lax.fori_loop(0, nchunks, chunk, 0)

    pltpu.sync_copy(acc, out_hbm.at[b, pl.ds(jlo, BLK)])


def _merge_sc(ssrc, sdst, idx):
    mesh = plsc.VectorSubcoreMesh(core_axis_name="c", subcore_axis_name="s")
    f = pl.kernel(
        _sc_merge_body,
        out_type=jax.ShapeDtypeStruct((B, T, D), jnp.float32),
        mesh=mesh,
        scratch_types=[
            pltpu.VMEM((T + 16,), jnp.int32),
            pltpu.VMEM((LIST,), jnp.int32),
            pltpu.VMEM((LIST,), jnp.int32),
            pltpu.VMEM((CH, D), jnp.float32),
            pltpu.VMEM((BLK, D), jnp.float32),
        ],
    )
    return f(ssrc, sdst, idx)


def kernel(hidden_states, attention_mask, self_attention_scores):
    Bh, L, Dd = hidden_states.shape
    assert Bh == B and L == 2 * T and Dd == D
    ssrc, sdst, idx = _route_tc(hidden_states)
    dst_m = _merge_sc(ssrc, sdst, idx.reshape(B, T))
    preserved = jnp.concatenate([hidden_states[:, :1, :], dst_m], axis=1)
    mask = jnp.zeros((B, 1, 1, T + 1), dtype=self_attention_scores.dtype)
    return preserved, mask
